# Initial kernel scaffold; baseline (speedup 1.0000x reference)
import jax
import jax.numpy as jnp
from jax.experimental import pallas as pl


def kernel(locations, logits_pred, reg_pred, ctrness_pred, det_cofs, basic_masks):
    raise NotImplementedError("write your pallas kernel here")


# speedup vs baseline: 2.9369x; 2.9369x over previous
import jax
import jax.numpy as jnp
from jax.experimental import pallas as pl
from jax.experimental.pallas import tpu as pltpu

_PRE_T = 0.05
_TOPK = 1000
_P = 1024
_NMS_T = 0.6
_POST = 100
_C = 80
_STRIDE = 8.0
_H, _W = 100, 168
_IMG_H, _IMG_W = 800.0, 1344.0
_NB = 32
_N = _H * _W
_OFF = 1346.0


def _scores_body(log_ref, ctr_ref, out_ref):
    s = jax.nn.sigmoid(log_ref[...])
    c = jax.nn.sigmoid(ctr_ref[...])
    out_ref[...] = jnp.where(s > _PRE_T, s * c, 0.0)


def _main_body(table_ref, ts_ref, tis_ref, tiv_ref,
               dets_ref, cls_ref, cof_ref,
               gath_ref, iou_ref):
    def gat(i, carry):
        j = tis_ref[i] // _C
        gath_ref[pl.ds(i, 1), :] = table_ref[pl.ds(j, 1), :]
        return carry
    jax.lax.fori_loop(0, _P, gat, 0)

    g = gath_ref[...]
    lx = g[:, 4:5]
    ly = g[:, 5:6]
    x1 = jnp.clip(lx - g[:, 0:1] * _STRIDE, 0.0, _IMG_W - 1.0)
    y1 = jnp.clip(ly - g[:, 1:2] * _STRIDE, 0.0, _IMG_H - 1.0)
    x2 = jnp.clip(lx + g[:, 2:3] * _STRIDE, 0.0, _IMG_W - 1.0)
    y2 = jnp.clip(ly + g[:, 3:4] * _STRIDE, 0.0, _IMG_H - 1.0)
    clsf = (tiv_ref[...] % _C).astype(jnp.float32)
    off = clsf * _OFF
    ox1 = x1 + off
    oy1 = y1 + off
    ox2 = x2 + off
    oy2 = y2 + off
    area = jnp.maximum(ox2 - ox1, 0.0) * jnp.maximum(oy2 - oy1, 0.0)

    tx1 = jnp.transpose(ox1)
    ty1 = jnp.transpose(oy1)
    tx2 = jnp.transpose(ox2)
    ty2 = jnp.transpose(oy2)
    tarea = jnp.transpose(area)

    for k in range(_P // 128):
        r0 = k * 128
        cx1 = ox1[r0:r0 + 128, :]
        cy1 = oy1[r0:r0 + 128, :]
        cx2 = ox2[r0:r0 + 128, :]
        cy2 = oy2[r0:r0 + 128, :]
        car = area[r0:r0 + 128, :]
        ix1 = jnp.maximum(cx1, tx1)
        iy1 = jnp.maximum(cy1, ty1)
        ix2 = jnp.minimum(cx2, tx2)
        iy2 = jnp.minimum(cy2, ty2)
        inter = jnp.maximum(ix2 - ix1, 0.0) * jnp.maximum(iy2 - iy1, 0.0)
        union = car + tarea - inter
        iou_ref[r0:r0 + 128, :] = inter / jnp.maximum(union, 1e-6)

    lin = jax.lax.broadcasted_iota(jnp.int32, (1, _P), 1)

    def nms_body(i, keepf):
        row = iou_ref[pl.ds(i, 1), :]
        ki = jnp.max(jnp.where(lin == i, keepf, 0.0))
        sup = (row > _NMS_T) & (lin > i) & (ki > 0.5)
        return jnp.where(sup, 0.0, keepf)

    keepr = jax.lax.fori_loop(0, _TOPK, nms_body,
                              jnp.ones((1, _P), jnp.float32))
    keepc = jnp.transpose(keepr)

    validc = (jax.lax.broadcasted_iota(jnp.int32, (_P, 1), 0)
              < _TOPK).astype(jnp.float32)
    keepv = keepc * validc
    unk = validc * (1.0 - keepc)
    i0 = jax.lax.broadcasted_iota(jnp.int32, (_P, _P), 0)
    i1 = jax.lax.broadcasted_iota(jnp.int32, (_P, _P), 1)
    below = (i1 < i0).astype(jnp.float32)
    posk = jnp.dot(below, keepv, preferred_element_type=jnp.float32,
                   precision=jax.lax.Precision.HIGHEST)
    posu = jnp.dot(below, unk, preferred_element_type=jnp.float32,
                   precision=jax.lax.Precision.HIGHEST)
    nkept = jnp.sum(keepv)
    pos = jnp.where(keepv > 0.0, posk, nkept + posu)
    pos = jnp.where(validc > 0.0, pos, 9999.0)
    posi = pos.astype(jnp.int32)

    r_iota = jax.lax.broadcasted_iota(jnp.int32, (_P, 128), 1)
    sel = (r_iota == posi).astype(jnp.float32)

    dsc = jnp.sqrt(jnp.maximum(ts_ref[...], 1e-12))
    scol = jnp.where(keepv > 0.0, dsc, -1.0)
    vals = jnp.concatenate([x1, y1, x2, y2, scol, clsf, g[:, 6:38]], axis=1)
    gathered = jax.lax.dot_general(
        sel, vals, (((0,), (0,)), ((), ())),
        preferred_element_type=jnp.float32,
        precision=jax.lax.Precision.HIGHEST)
    dets_ref[...] = gathered[0:_POST, 0:5]
    cls_ref[...] = jnp.round(gathered[0:_POST, 5:6]).astype(jnp.int32)
    cof_ref[...] = gathered[0:_POST, 6:38]


def _mask_body(cof_ref, dets_ref, basic_ref, xs_ref, ys_ref, out_ref):
    mm = jnp.dot(cof_ref[...], basic_ref[...],
                 preferred_element_type=jnp.float32,
                 precision=jax.lax.Precision.HIGHEST)
    sig = jax.nn.sigmoid(mm)
    x1 = dets_ref[:, 0:1]
    y1 = dets_ref[:, 1:2]
    x2 = dets_ref[:, 2:3]
    y2 = dets_ref[:, 3:4]
    xs = xs_ref[...]
    ys = ys_ref[...]
    inside = (xs >= x1) & (xs <= x2) & (ys >= y1) & (ys <= y2)
    out_ref[...] = jnp.where(inside, sig, 0.0)


def kernel(locations, logits_pred, reg_pred, ctrness_pred, det_cofs,
           basic_masks):
    f32 = jnp.float32
    scores = pl.pallas_call(
        _scores_body,
        out_shape=jax.ShapeDtypeStruct((_N, _C), f32),
    )(logits_pred, ctrness_pred.reshape(_N, 1))

    ts, ti = jax.lax.top_k(scores.reshape(-1), _TOPK)
    tsp = jnp.pad(ts, (0, _P - _TOPK)).reshape(_P, 1)
    tip = jnp.pad(ti, (0, _P - _TOPK))
    table = jnp.concatenate([reg_pred, locations, det_cofs], axis=1)

    dets, cls2, cof = pl.pallas_call(
        _main_body,
        out_shape=(
            jax.ShapeDtypeStruct((_POST, 5), f32),
            jax.ShapeDtypeStruct((_POST, 1), jnp.int32),
            jax.ShapeDtypeStruct((_POST, _NB), f32),
        ),
        in_specs=[
            pl.BlockSpec(memory_space=pltpu.VMEM),
            pl.BlockSpec(memory_space=pltpu.VMEM),
            pl.BlockSpec(memory_space=pltpu.SMEM),
            pl.BlockSpec(memory_space=pltpu.VMEM),
        ],
        scratch_shapes=[
            pltpu.VMEM((_P, 38), f32),
            pltpu.VMEM((_P, _P), f32),
        ],
    )(table, tsp, tip, tip.reshape(_P, 1))

    xs1 = (jnp.arange(_W, dtype=f32) + 0.5) * _STRIDE
    ys1 = (jnp.arange(_H, dtype=f32) + 0.5) * _STRIDE
    xsf = jnp.tile(xs1, _H).reshape(1, _N)
    ysf = jnp.repeat(ys1, _W).reshape(1, _N)
    masks = pl.pallas_call(
        _mask_body,
        out_shape=jax.ShapeDtypeStruct((_POST, _N), f32),
    )(cof, dets, basic_masks.reshape(_NB, _N), xsf, ysf)

    return dets, masks.reshape(_POST, _H, _W), cls2.reshape(_POST)
